# baseline (speedup 1.0000x reference)
import functools

import jax
import jax.numpy as jnp
from jax import lax
from jax.experimental import pallas as pl
from jax.experimental.pallas import tpu as pltpu
from jax.experimental.pallas import tpu_sc as plsc

BATCH = 16384
EMBED = 32
LINE = 128
PACK = LINE // EMBED
NLINES = 250112
N_TR_BLOCKS = NLINES // LINE
NC = 2
NS = 16
NW = NC * NS
B_PER_W = BATCH // NW
CHUNK = 128
N_CHUNKS = B_PER_W // CHUNK
VLANES = 16


def _pack_body(in_ref, o_ref):
    x = in_ref[...]
    for k in range(PACK):
        o_ref[:, k * EMBED:(k + 1) * EMBED] = x[:, k * LINE:(k + 1) * LINE].T


def _pack(t_t):
    return pl.pallas_call(
        _pack_body,
        grid=(N_TR_BLOCKS,),
        in_specs=[pl.BlockSpec((EMBED, PACK * LINE), lambda i: (0, i))],
        out_specs=pl.BlockSpec((LINE, LINE), lambda i: (i, 0)),
        out_shape=jax.ShapeDtypeStruct((NLINES, LINE), jnp.float32),
    )(t_t)


def _sc_gather_body(user_p, business_p, uid, bid, out_u, out_b,
                    idx, rows, sem):
    wid = lax.axis_index("c") * NS + lax.axis_index("s")
    base = wid * B_PER_W
    for ids_hbm, table, out in ((uid, user_p, out_u),
                                (bid, business_p, out_b)):
        pltpu.sync_copy(ids_hbm.at[pl.ds(base, B_PER_W)], idx)
        for k in range(B_PER_W // VLANES):
            sl = pl.ds(k * VLANES, VLANES)
            v = idx[sl]
            hi = lax.shift_left(lax.shift_right_logical(v, 9), 7)
            idx[sl] = hi | (v & (LINE - 1))
        copies = []
        for c in range(N_CHUNKS):
            sl = pl.ds(c * CHUNK, CHUNK)
            copies.append(pltpu.async_copy(
                table.at[idx.at[sl]], rows.at[sl], sem))
        for cp in copies:
            cp.wait()
        pltpu.sync_copy(rows, out.at[pl.ds(base, B_PER_W)])


@functools.cache
def _sc_gather():
    return pl.kernel(
        _sc_gather_body,
        out_type=[
            jax.ShapeDtypeStruct((BATCH, LINE), jnp.float32),
            jax.ShapeDtypeStruct((BATCH, LINE), jnp.float32),
        ],
        mesh=plsc.VectorSubcoreMesh(core_axis_name="c", subcore_axis_name="s"),
        scratch_types=[
            pltpu.VMEM((B_PER_W,), jnp.int32),
            pltpu.VMEM((B_PER_W, LINE), jnp.float32),
            pltpu.SemaphoreType.DMA,
        ],
    )


def _mlp_body(uid_ref, bid_ref, ur_ref, br_ref, w1u_ref, w1b_ref, b1_ref,
              w2_ref, b2_ref, o_ref):
    uoff = lax.shift_right_logical(uid_ref[...], 7) & (PACK - 1)
    boff = lax.shift_right_logical(bid_ref[...], 7) & (PACK - 1)
    ur = ur_ref[...]
    br = br_ref[...]
    usel = jnp.zeros_like(ur[:, :EMBED])
    bsel = jnp.zeros_like(usel)
    for k in range(PACK):
        usel += jnp.where(uoff == k, ur[:, k * EMBED:(k + 1) * EMBED], 0.0)
        bsel += jnp.where(boff == k, br[:, k * EMBED:(k + 1) * EMBED], 0.0)
    h = (jnp.dot(usel, w1u_ref[...], preferred_element_type=jnp.float32)
         + jnp.dot(bsel, w1b_ref[...], preferred_element_type=jnp.float32)
         + b1_ref[...])
    h = jnp.maximum(h, 0.0)
    o = jnp.dot(h, w2_ref[...], preferred_element_type=jnp.float32) + b2_ref[...]
    o_ref[...] = jnp.maximum(o, 0.0).T


def _mlp(uid2, bid2, u_rows, b_rows, w1u_t, w1b_t, b1, w2_t, b2, block=2048):
    n_blocks = BATCH // block
    return pl.pallas_call(
        _mlp_body,
        grid=(n_blocks,),
        in_specs=[
            pl.BlockSpec((block, 1), lambda i: (i, 0)),
            pl.BlockSpec((block, 1), lambda i: (i, 0)),
            pl.BlockSpec((block, LINE), lambda i: (i, 0)),
            pl.BlockSpec((block, LINE), lambda i: (i, 0)),
            pl.BlockSpec(w1u_t.shape, lambda i: (0, 0)),
            pl.BlockSpec(w1b_t.shape, lambda i: (0, 0)),
            pl.BlockSpec(b1.shape, lambda i: (0, 0)),
            pl.BlockSpec(w2_t.shape, lambda i: (0, 0)),
            pl.BlockSpec(b2.shape, lambda i: (0, 0)),
        ],
        out_specs=pl.BlockSpec((w2_t.shape[1], block), lambda i: (0, i)),
        out_shape=jax.ShapeDtypeStruct((w2_t.shape[1], BATCH), jnp.float32),
    )(uid2, bid2, u_rows, b_rows, w1u_t, w1b_t, b1, w2_t, b2)


def kernel(user_ids, business_ids, user_table, business_table, W1, b1, W2, b2):
    uid = user_ids.astype(jnp.int32)
    bid = business_ids.astype(jnp.int32)
    user_p = _pack(user_table.T)
    business_p = _pack(business_table.T)
    u_rows, b_rows = _sc_gather()(user_p, business_p, uid, bid)
    w1u_t = W1[:, :EMBED].T
    w1b_t = W1[:, EMBED:].T
    w2_t = W2.T
    o_t = _mlp(uid.reshape(-1, 1), bid.reshape(-1, 1), u_rows, b_rows,
               w1u_t, w1b_t, b1.reshape(1, -1), w2_t, b2.reshape(1, -1))
    return o_t.T


# speedup vs baseline: 3.6144x; 3.6144x over previous
import functools

import jax
import jax.numpy as jnp
from jax import lax
from jax.experimental import pallas as pl
from jax.experimental.pallas import tpu as pltpu
from jax.experimental.pallas import tpu_sc as plsc

BATCH = 16384
EMBED = 32
LINE = 128
PACK = LINE // EMBED
TR_BLK = 4096
N_TR_BLOCKS = 245
NLINES = N_TR_BLOCKS * (TR_BLK // PACK)
NC = 2
NS = 16
NW = NC * NS
B_PER_W = BATCH // NW
CHUNK = 128
N_CHUNKS = B_PER_W // CHUNK
VLANES = 16


def _pack_body(in_ref, o_ref):
    x = in_ref[...]
    eye = jnp.eye(EMBED, dtype=jnp.float32)
    y = lax.dot_general(x, eye, (((0,), (0,)), ((), ())),
                        preferred_element_type=jnp.float32)
    for s in range(TR_BLK // 512):
        for k in range(PACK):
            o_ref[s * LINE:(s + 1) * LINE, k * EMBED:(k + 1) * EMBED] = (
                y[s * 512 + k * LINE:s * 512 + (k + 1) * LINE, :])


def _pack(t_t):
    return pl.pallas_call(
        _pack_body,
        grid=(N_TR_BLOCKS,),
        in_specs=[pl.BlockSpec((EMBED, TR_BLK), lambda i: (0, i))],
        out_specs=pl.BlockSpec((TR_BLK // PACK, LINE), lambda i: (i, 0)),
        out_shape=jax.ShapeDtypeStruct((NLINES, LINE), jnp.float32),
    )(t_t)


def _sc_gather_body(user_p, business_p, uid, bid, out_u, out_b,
                    idx, rows, sem):
    wid = lax.axis_index("c") * NS + lax.axis_index("s")
    base = wid * B_PER_W
    for ids_hbm, table, out in ((uid, user_p, out_u),
                                (bid, business_p, out_b)):
        pltpu.sync_copy(ids_hbm.at[pl.ds(base, B_PER_W)], idx)
        for k in range(B_PER_W // VLANES):
            sl = pl.ds(k * VLANES, VLANES)
            v = idx[sl]
            hi = lax.shift_left(lax.shift_right_logical(v, 9), 7)
            idx[sl] = hi | (v & (LINE - 1))
        copies = []
        for c in range(N_CHUNKS):
            sl = pl.ds(c * CHUNK, CHUNK)
            copies.append(pltpu.async_copy(
                table.at[idx.at[sl]], rows.at[sl], sem))
        for cp in copies:
            cp.wait()
        pltpu.sync_copy(rows, out.at[pl.ds(base, B_PER_W)])


@functools.cache
def _sc_gather():
    return pl.kernel(
        _sc_gather_body,
        out_type=[
            jax.ShapeDtypeStruct((BATCH, LINE), jnp.float32),
            jax.ShapeDtypeStruct((BATCH, LINE), jnp.float32),
        ],
        mesh=plsc.VectorSubcoreMesh(core_axis_name="c", subcore_axis_name="s"),
        scratch_types=[
            pltpu.VMEM((B_PER_W,), jnp.int32),
            pltpu.VMEM((B_PER_W, LINE), jnp.float32),
            pltpu.SemaphoreType.DMA,
        ],
    )


def _mlp_body(uid_ref, bid_ref, ur_ref, br_ref, w1u_ref, w1b_ref, b1_ref,
              w2_ref, b2_ref, o_ref):
    uoff = lax.shift_right_logical(uid_ref[...], 7) & (PACK - 1)
    boff = lax.shift_right_logical(bid_ref[...], 7) & (PACK - 1)
    ur = ur_ref[...]
    br = br_ref[...]
    usel = jnp.zeros_like(ur[:, :EMBED])
    bsel = jnp.zeros_like(usel)
    for k in range(PACK):
        usel += jnp.where(uoff == k, ur[:, k * EMBED:(k + 1) * EMBED], 0.0)
        bsel += jnp.where(boff == k, br[:, k * EMBED:(k + 1) * EMBED], 0.0)
    h = (jnp.dot(usel, w1u_ref[...], preferred_element_type=jnp.float32)
         + jnp.dot(bsel, w1b_ref[...], preferred_element_type=jnp.float32)
         + b1_ref[...])
    h = jnp.maximum(h, 0.0)
    o = jnp.dot(h, w2_ref[...], preferred_element_type=jnp.float32) + b2_ref[...]
    o_ref[...] = jnp.maximum(o, 0.0).T


def _mlp(uid2, bid2, u_rows, b_rows, w1u_t, w1b_t, b1, w2_t, b2, block=2048):
    n_blocks = BATCH // block
    return pl.pallas_call(
        _mlp_body,
        grid=(n_blocks,),
        in_specs=[
            pl.BlockSpec((block, 1), lambda i: (i, 0)),
            pl.BlockSpec((block, 1), lambda i: (i, 0)),
            pl.BlockSpec((block, LINE), lambda i: (i, 0)),
            pl.BlockSpec((block, LINE), lambda i: (i, 0)),
            pl.BlockSpec(w1u_t.shape, lambda i: (0, 0)),
            pl.BlockSpec(w1b_t.shape, lambda i: (0, 0)),
            pl.BlockSpec(b1.shape, lambda i: (0, 0)),
            pl.BlockSpec(w2_t.shape, lambda i: (0, 0)),
            pl.BlockSpec(b2.shape, lambda i: (0, 0)),
        ],
        out_specs=pl.BlockSpec((w2_t.shape[1], block), lambda i: (0, i)),
        out_shape=jax.ShapeDtypeStruct((w2_t.shape[1], BATCH), jnp.float32),
    )(uid2, bid2, u_rows, b_rows, w1u_t, w1b_t, b1, w2_t, b2)


def kernel(user_ids, business_ids, user_table, business_table, W1, b1, W2, b2):
    uid = user_ids.astype(jnp.int32)
    bid = business_ids.astype(jnp.int32)
    user_p = _pack(user_table.T)
    business_p = _pack(business_table.T)
    u_rows, b_rows = _sc_gather()(user_p, business_p, uid, bid)
    w1u_t = W1[:, :EMBED].T
    w1b_t = W1[:, EMBED:].T
    w2_t = W2.T
    o_t = _mlp(uid.reshape(-1, 1), bid.reshape(-1, 1), u_rows, b_rows,
               w1u_t, w1b_t, b1.reshape(1, -1), w2_t, b2.reshape(1, -1))
    return o_t.T


# speedup vs baseline: 4.1777x; 1.1558x over previous
import functools

import jax
import jax.numpy as jnp
from jax import lax
from jax.experimental import pallas as pl
from jax.experimental.pallas import tpu as pltpu
from jax.experimental.pallas import tpu_sc as plsc

BATCH = 16384
EMBED = 32
LINE = 128
PACK = LINE // EMBED
TR_BLK = 8192
N_TR_BLOCKS = 123
NLINES = N_TR_BLOCKS * (TR_BLK // PACK)
NC = 2
NS = 16
NW = NC * NS
B_PER_W = BATCH // NW
CHUNK = 128
N_CHUNKS = B_PER_W // CHUNK
VLANES = 16


def _pack_body(in_ref, o_ref):
    x = in_ref[...]
    eye = jnp.eye(EMBED, dtype=jnp.float32)
    y = lax.dot_general(x, eye, (((0,), (0,)), ((), ())),
                        preferred_element_type=jnp.float32)
    for s in range(TR_BLK // 512):
        o_ref[s * LINE:(s + 1) * LINE, :] = jnp.concatenate(
            [y[s * 512 + k * LINE:s * 512 + (k + 1) * LINE, :]
             for k in range(PACK)], axis=1)


def _pack(t_t):
    return pl.pallas_call(
        _pack_body,
        grid=(N_TR_BLOCKS,),
        in_specs=[pl.BlockSpec((EMBED, TR_BLK), lambda i: (0, i))],
        out_specs=pl.BlockSpec((TR_BLK // PACK, LINE), lambda i: (i, 0)),
        out_shape=jax.ShapeDtypeStruct((NLINES, LINE), jnp.float32),
    )(t_t)


def _sc_gather_body(user_p, business_p, uid, bid, out_u, out_b,
                    idx, rows, sem):
    wid = lax.axis_index("c") * NS + lax.axis_index("s")
    base = wid * B_PER_W
    for ids_hbm, table, out in ((uid, user_p, out_u),
                                (bid, business_p, out_b)):
        pltpu.sync_copy(ids_hbm.at[pl.ds(base, B_PER_W)], idx)
        for k in range(B_PER_W // VLANES):
            sl = pl.ds(k * VLANES, VLANES)
            v = idx[sl]
            hi = lax.shift_left(lax.shift_right_logical(v, 9), 7)
            idx[sl] = hi | (v & (LINE - 1))
        copies = []
        for c in range(N_CHUNKS):
            sl = pl.ds(c * CHUNK, CHUNK)
            copies.append(pltpu.async_copy(
                table.at[idx.at[sl]], rows.at[sl], sem))
        for cp in copies:
            cp.wait()
        pltpu.sync_copy(rows, out.at[pl.ds(base, B_PER_W)])


@functools.cache
def _sc_gather():
    return pl.kernel(
        _sc_gather_body,
        out_type=[
            jax.ShapeDtypeStruct((BATCH, LINE), jnp.float32),
            jax.ShapeDtypeStruct((BATCH, LINE), jnp.float32),
        ],
        mesh=plsc.VectorSubcoreMesh(core_axis_name="c", subcore_axis_name="s"),
        scratch_types=[
            pltpu.VMEM((B_PER_W,), jnp.int32),
            pltpu.VMEM((B_PER_W, LINE), jnp.float32),
            pltpu.SemaphoreType.DMA,
        ],
    )


def _mlp_body(uid_ref, bid_ref, ur_ref, br_ref, w1u_ref, w1b_ref, b1_ref,
              w2_ref, b2_ref, o_ref):
    uoff = lax.shift_right_logical(uid_ref[...], 7) & (PACK - 1)
    boff = lax.shift_right_logical(bid_ref[...], 7) & (PACK - 1)
    ur = ur_ref[...]
    br = br_ref[...]
    usel = jnp.zeros_like(ur[:, :EMBED])
    bsel = jnp.zeros_like(usel)
    for k in range(PACK):
        usel += jnp.where(uoff == k, ur[:, k * EMBED:(k + 1) * EMBED], 0.0)
        bsel += jnp.where(boff == k, br[:, k * EMBED:(k + 1) * EMBED], 0.0)
    h = (jnp.dot(usel, w1u_ref[...], preferred_element_type=jnp.float32)
         + jnp.dot(bsel, w1b_ref[...], preferred_element_type=jnp.float32)
         + b1_ref[...])
    h = jnp.maximum(h, 0.0)
    o = jnp.dot(h, w2_ref[...], preferred_element_type=jnp.float32) + b2_ref[...]
    o_ref[...] = jnp.maximum(o, 0.0).T


def _mlp(uid2, bid2, u_rows, b_rows, w1u_t, w1b_t, b1, w2_t, b2, block=2048):
    n_blocks = BATCH // block
    return pl.pallas_call(
        _mlp_body,
        grid=(n_blocks,),
        in_specs=[
            pl.BlockSpec((block, 1), lambda i: (i, 0)),
            pl.BlockSpec((block, 1), lambda i: (i, 0)),
            pl.BlockSpec((block, LINE), lambda i: (i, 0)),
            pl.BlockSpec((block, LINE), lambda i: (i, 0)),
            pl.BlockSpec(w1u_t.shape, lambda i: (0, 0)),
            pl.BlockSpec(w1b_t.shape, lambda i: (0, 0)),
            pl.BlockSpec(b1.shape, lambda i: (0, 0)),
            pl.BlockSpec(w2_t.shape, lambda i: (0, 0)),
            pl.BlockSpec(b2.shape, lambda i: (0, 0)),
        ],
        out_specs=pl.BlockSpec((w2_t.shape[1], block), lambda i: (0, i)),
        out_shape=jax.ShapeDtypeStruct((w2_t.shape[1], BATCH), jnp.float32),
    )(uid2, bid2, u_rows, b_rows, w1u_t, w1b_t, b1, w2_t, b2)


def kernel(user_ids, business_ids, user_table, business_table, W1, b1, W2, b2):
    uid = user_ids.astype(jnp.int32)
    bid = business_ids.astype(jnp.int32)
    user_p = _pack(user_table.T)
    business_p = _pack(business_table.T)
    u_rows, b_rows = _sc_gather()(user_p, business_p, uid, bid)
    w1u_t = W1[:, :EMBED].T
    w1b_t = W1[:, EMBED:].T
    w2_t = W2.T
    o_t = _mlp(uid.reshape(-1, 1), bid.reshape(-1, 1), u_rows, b_rows,
               w1u_t, w1b_t, b1.reshape(1, -1), w2_t, b2.reshape(1, -1))
    return o_t.T


# speedup vs baseline: 4.1926x; 1.0036x over previous
import functools

import jax
import jax.numpy as jnp
from jax import lax
from jax.experimental import pallas as pl
from jax.experimental.pallas import tpu as pltpu
from jax.experimental.pallas import tpu_sc as plsc

BATCH = 16384
EMBED = 32
LINE = 128
PACK = LINE // EMBED
TR_BLK = 8192
N_TR_BLOCKS = 123
NLINES = N_TR_BLOCKS * (TR_BLK // PACK)
NC = 2
NS = 16
NW = NC * NS
B_PER_W = BATCH // NW
CHUNK = 128
N_CHUNKS = B_PER_W // CHUNK
VLANES = 16


def _pack_body(in_ref, o_ref):
    x = in_ref[...]
    y = x.T
    for s in range(TR_BLK // 512):
        o_ref[s * LINE:(s + 1) * LINE, :] = jnp.concatenate(
            [y[s * 512 + k * LINE:s * 512 + (k + 1) * LINE, :]
             for k in range(PACK)], axis=1)


def _pack(t_t):
    return pl.pallas_call(
        _pack_body,
        grid=(N_TR_BLOCKS,),
        in_specs=[pl.BlockSpec((EMBED, TR_BLK), lambda i: (0, i))],
        out_specs=pl.BlockSpec((TR_BLK // PACK, LINE), lambda i: (i, 0)),
        out_shape=jax.ShapeDtypeStruct((NLINES, LINE), jnp.float32),
    )(t_t)


def _sc_gather_body(user_p, business_p, uid, bid, out_u, out_b,
                    idx, rows, sem):
    wid = lax.axis_index("c") * NS + lax.axis_index("s")
    base = wid * B_PER_W
    for ids_hbm, table, out in ((uid, user_p, out_u),
                                (bid, business_p, out_b)):
        pltpu.sync_copy(ids_hbm.at[pl.ds(base, B_PER_W)], idx)
        for k in range(B_PER_W // VLANES):
            sl = pl.ds(k * VLANES, VLANES)
            v = idx[sl]
            hi = lax.shift_left(lax.shift_right_logical(v, 9), 7)
            idx[sl] = hi | (v & (LINE - 1))
        copies = []
        for c in range(N_CHUNKS):
            sl = pl.ds(c * CHUNK, CHUNK)
            copies.append(pltpu.async_copy(
                table.at[idx.at[sl]], rows.at[sl], sem))
        for cp in copies:
            cp.wait()
        pltpu.sync_copy(rows, out.at[pl.ds(base, B_PER_W)])


@functools.cache
def _sc_gather():
    return pl.kernel(
        _sc_gather_body,
        out_type=[
            jax.ShapeDtypeStruct((BATCH, LINE), jnp.float32),
            jax.ShapeDtypeStruct((BATCH, LINE), jnp.float32),
        ],
        mesh=plsc.VectorSubcoreMesh(core_axis_name="c", subcore_axis_name="s"),
        scratch_types=[
            pltpu.VMEM((B_PER_W,), jnp.int32),
            pltpu.VMEM((B_PER_W, LINE), jnp.float32),
            pltpu.SemaphoreType.DMA,
        ],
    )


def _mlp_body(uid_ref, bid_ref, ur_ref, br_ref, w1u_ref, w1b_ref, b1_ref,
              w2_ref, b2_ref, o_ref):
    uoff = lax.shift_right_logical(uid_ref[...], 7) & (PACK - 1)
    boff = lax.shift_right_logical(bid_ref[...], 7) & (PACK - 1)
    ur = ur_ref[...]
    br = br_ref[...]
    usel = jnp.zeros_like(ur[:, :EMBED])
    bsel = jnp.zeros_like(usel)
    for k in range(PACK):
        usel += jnp.where(uoff == k, ur[:, k * EMBED:(k + 1) * EMBED], 0.0)
        bsel += jnp.where(boff == k, br[:, k * EMBED:(k + 1) * EMBED], 0.0)
    h = (jnp.dot(usel, w1u_ref[...], preferred_element_type=jnp.float32)
         + jnp.dot(bsel, w1b_ref[...], preferred_element_type=jnp.float32)
         + b1_ref[...])
    h = jnp.maximum(h, 0.0)
    o = jnp.dot(h, w2_ref[...], preferred_element_type=jnp.float32) + b2_ref[...]
    o_ref[...] = jnp.maximum(o, 0.0).T


def _mlp(uid2, bid2, u_rows, b_rows, w1u_t, w1b_t, b1, w2_t, b2, block=2048):
    n_blocks = BATCH // block
    return pl.pallas_call(
        _mlp_body,
        grid=(n_blocks,),
        in_specs=[
            pl.BlockSpec((block, 1), lambda i: (i, 0)),
            pl.BlockSpec((block, 1), lambda i: (i, 0)),
            pl.BlockSpec((block, LINE), lambda i: (i, 0)),
            pl.BlockSpec((block, LINE), lambda i: (i, 0)),
            pl.BlockSpec(w1u_t.shape, lambda i: (0, 0)),
            pl.BlockSpec(w1b_t.shape, lambda i: (0, 0)),
            pl.BlockSpec(b1.shape, lambda i: (0, 0)),
            pl.BlockSpec(w2_t.shape, lambda i: (0, 0)),
            pl.BlockSpec(b2.shape, lambda i: (0, 0)),
        ],
        out_specs=pl.BlockSpec((w2_t.shape[1], block), lambda i: (0, i)),
        out_shape=jax.ShapeDtypeStruct((w2_t.shape[1], BATCH), jnp.float32),
    )(uid2, bid2, u_rows, b_rows, w1u_t, w1b_t, b1, w2_t, b2)


def kernel(user_ids, business_ids, user_table, business_table, W1, b1, W2, b2):
    uid = user_ids.astype(jnp.int32)
    bid = business_ids.astype(jnp.int32)
    user_p = _pack(user_table.T)
    business_p = _pack(business_table.T)
    u_rows, b_rows = _sc_gather()(user_p, business_p, uid, bid)
    w1u_t = W1[:, :EMBED].T
    w1b_t = W1[:, EMBED:].T
    w2_t = W2.T
    o_t = _mlp(uid.reshape(-1, 1), bid.reshape(-1, 1), u_rows, b_rows,
               w1u_t, w1b_t, b1.reshape(1, -1), w2_t, b2.reshape(1, -1))
    return o_t.T
